# baseline (speedup 1.0000x reference)
import functools

import jax
import jax.numpy as jnp
from jax import lax
from jax.experimental import pallas as pl
from jax.experimental.pallas import tpu as pltpu
from jax.experimental.pallas import tpu_sc as plsc

F = 128
NE = 64
MSG = 64
HEADS = 2
TAU = 0.1
N_REAL = 10000
E_REAL = 320000

NP = 10240
CE = 4096
NCHUNK = 80
EP = NCHUNK * CE
NT = 32
ET = EP // NT
KT = ET // 128
ZR = NP // 16

_f32 = jnp.float32
_HI = lax.Precision.HIGHEST


def _leaky(v):
    return jnp.where(v >= 0, v, 0.01 * v)


def _onehot(dst, n_rows):
    return (dst == lax.broadcasted_iota(jnp.int32, (n_rows, F), 1)).astype(_f32)


def _dot(a, b):
    return jnp.dot(a, b, preferred_element_type=_f32)


def _doth(a, b):
    return jnp.dot(a, b, preferred_element_type=_f32, precision=_HI)


def _blk0_body(dst_ref, ev_ref, wm1f_ref, wef_ref, rows_ref, m1_ref, e1_ref,
               aggf_ref):
    i = pl.program_id(0)
    dst = dst_ref[...]
    ev = ev_ref[...]
    oh = _onehot(dst, CE)
    rows = rows_ref[...]
    w1, w2, we, c_m2, c_e = (rows[0:1], rows[1:2], rows[2:3], rows[3:4],
                             rows[4:5])
    m1 = jnp.maximum(_dot(oh, wm1f_ref[...]) + ev * w1, 0.0)
    m1_ref[...] = m1
    e1 = c_e + _dot(oh, wef_ref[...]) + ev * we
    e1_ref[...] = _leaky(e1)
    m2 = jnp.maximum(c_m2 + ev * w2, 0.0)
    gidx = i * CE + lax.broadcasted_iota(jnp.int32, (CE, 1), 0)
    m2 = jnp.where(gidx < E_REAL, m2, 0.0)
    part = lax.dot_general(oh, m2, (((0,), (0,)), ((), ())),
                           preferred_element_type=_f32, precision=_HI)

    @pl.when(i == 0)
    def _():
        aggf_ref[...] = jnp.zeros_like(aggf_ref)

    aggf_ref[...] += part


def _blk1_body(dst_ref, e1_ref, g_ref, feat_ref, wm1t_ref, wm1b_ref, wm2t_ref,
               wm2b_ref, wet_ref, wef_ref, web_ref, m1_ref, e2_ref, aggf_ref):
    i = pl.program_id(0)
    dst = dst_ref[...]
    e1 = e1_ref[...]
    g = g_ref[...]
    oh = _onehot(dst, CE)
    featd = _doth(oh, feat_ref[...])
    m1 = jnp.maximum(_dot(featd, wm1t_ref[...]) + _dot(e1, wm1b_ref[...]), 0.0)
    m1_ref[...] = m1
    m2 = jnp.maximum(_dot(g, wm2t_ref[...]) + _dot(e1, wm2b_ref[...]), 0.0)
    e2 = _dot(g, wet_ref[...]) + _dot(featd, wef_ref[...]) \
        + _dot(e1, web_ref[...])
    e2_ref[...] = _leaky(e2)
    gidx = i * CE + lax.broadcasted_iota(jnp.int32, (CE, 1), 0)
    m2 = jnp.where(gidx < E_REAL, m2, 0.0)
    part = lax.dot_general(oh, m2, (((0,), (0,)), ((), ())),
                           preferred_element_type=_f32, precision=_HI)

    @pl.when(i == 0)
    def _():
        aggf_ref[...] = jnp.zeros_like(aggf_ref)

    aggf_ref[...] += part


def _blk2_body(dst_ref, e2_ref, g2_ref, feat_ref, wm1t_ref, wm1b_ref,
               wm2t_ref, wm2b_ref, m1_ref, aggf_ref):
    i = pl.program_id(0)
    dst = dst_ref[...]
    e2 = e2_ref[...]
    oh = _onehot(dst, CE)
    featd = _doth(oh, feat_ref[...])
    m1 = jnp.maximum(_dot(featd, wm1t_ref[...]) + _dot(e2, wm1b_ref[...]), 0.0)
    m1_ref[...] = m1
    m2 = jnp.maximum(_dot(g2_ref[...], wm2t_ref[...])
                     + _dot(e2, wm2b_ref[...]), 0.0)
    gidx = i * CE + lax.broadcasted_iota(jnp.int32, (CE, 1), 0)
    m2 = jnp.where(gidx < E_REAL, m2, 0.0)
    part = lax.dot_general(oh, m2, (((0,), (0,)), ((), ())),
                           preferred_element_type=_f32, precision=_HI)

    @pl.when(i == 0)
    def _():
        aggf_ref[...] = jnp.zeros_like(aggf_ref)

    aggf_ref[...] += part


def _edge_chunk_call(body, chunk_shapes, const_shapes, out_shapes,
                     out_chunked):
    in_specs = []
    for shp in chunk_shapes:
        in_specs.append(pl.BlockSpec((CE, shp), lambda i: (i, 0)))
    for shp in const_shapes:
        in_specs.append(
            pl.BlockSpec(shp, lambda i, _z=tuple(0 for _ in shp): _z))
    out_specs = []
    for shp, chunked in zip(out_shapes, out_chunked):
        if chunked:
            out_specs.append(pl.BlockSpec((CE, shp[1]), lambda i: (i, 0)))
        else:
            out_specs.append(pl.BlockSpec(shp, lambda i: (0, 0)))
    out_shape = [
        jax.ShapeDtypeStruct((EP, shp[1]) if chunked else shp, _f32)
        for shp, chunked in zip(out_shapes, out_chunked)
    ]
    return pl.pallas_call(
        body,
        grid=(NCHUNK,),
        in_specs=in_specs,
        out_specs=out_specs,
        out_shape=out_shape,
    )


def _sc_mesh():
    return plsc.VectorSubcoreMesh(core_axis_name="c", subcore_axis_name="s")


def _make_sc_gather(width):
    @functools.partial(
        pl.kernel,
        out_type=jax.ShapeDtypeStruct((EP, width), _f32),
        mesh=_sc_mesh(),
        compiler_params=pltpu.CompilerParams(use_tc_tiling_on_sc=False),
        scratch_types=[
            pltpu.VMEM((KT, 128), jnp.int32),
            pltpu.VMEM((128, width), _f32),
            pltpu.VMEM((128, width), _f32),
            pltpu.SemaphoreType.DMA,
            pltpu.SemaphoreType.DMA,
        ],
    )
    def gather(table_hbm, idx_hbm, out_hbm, idx_v, b0, b1, s0, s1):
        c = lax.axis_index("c")
        s = lax.axis_index("s")
        t = c * 16 + s
        base = t * KT
        pltpu.sync_copy(idx_hbm.at[pl.ds(base, KT)], idx_v)
        pltpu.async_copy(table_hbm.at[idx_v.at[0]], b0, s0)

        def body(h, carry):
            j0 = 2 * h
            d1 = pltpu.async_copy(table_hbm.at[idx_v.at[j0 + 1]], b1, s1)
            pltpu.make_async_copy(table_hbm.at[idx_v.at[j0]], b0, s0).wait()
            pltpu.sync_copy(b0, out_hbm.at[pl.ds((base + j0) * 128, 128)])

            @pl.when(h + 1 < KT // 2)
            def _():
                pltpu.async_copy(table_hbm.at[idx_v.at[j0 + 2]], b0, s0)

            d1.wait()
            pltpu.sync_copy(b1, out_hbm.at[pl.ds((base + j0 + 1) * 128, 128)])
            return carry

        lax.fori_loop(0, KT // 2, body, 0)

    return gather


def _make_sc_scatter():
    @functools.partial(
        pl.kernel,
        out_type=jax.ShapeDtypeStruct((2 * NP, MSG), _f32),
        mesh=_sc_mesh(),
        compiler_params=pltpu.CompilerParams(use_tc_tiling_on_sc=False),
        scratch_types=[
            pltpu.VMEM((KT, 128), jnp.int32),
            pltpu.VMEM((128, MSG), _f32),
            pltpu.VMEM((128, MSG), _f32),
            pltpu.VMEM_SHARED((NP, MSG), _f32),
            pltpu.SemaphoreType.DMA,
            pltpu.SemaphoreType.DMA,
        ],
    )
    def scatter(idx_hbm, vals_hbm, zeros_hbm, out_hbm, idx_v, v0, v1, acc,
                s0, s1):
        c = lax.axis_index("c")
        s = lax.axis_index("s")
        t = c * 16 + s
        pltpu.sync_copy(zeros_hbm.at[pl.ds(s * ZR, ZR)], acc.at[pl.ds(s * ZR, ZR)])
        plsc.subcore_barrier()
        base = t * KT
        pltpu.sync_copy(idx_hbm.at[pl.ds(base, KT)], idx_v)
        pltpu.async_copy(vals_hbm.at[pl.ds(base * 128, 128)], v0, s0)

        def body(h, carry):
            j0 = 2 * h
            d1 = pltpu.async_copy(vals_hbm.at[pl.ds((base + j0 + 1) * 128, 128)],
                                  v1, s1)
            pltpu.make_async_copy(vals_hbm.at[pl.ds(base * 128, 128)], v0,
                                  s0).wait()
            pltpu.sync_copy(v0, acc.at[idx_v.at[j0]], add=True)

            @pl.when(h + 1 < KT // 2)
            def _():
                pltpu.async_copy(vals_hbm.at[pl.ds((base + j0 + 2) * 128, 128)],
                                 v0, s0)

            d1.wait()
            pltpu.sync_copy(v1, acc.at[idx_v.at[j0 + 1]], add=True)
            return carry

        lax.fori_loop(0, KT // 2, body, 0)
        plsc.subcore_barrier()
        pltpu.sync_copy(acc.at[pl.ds(s * ZR, ZR)],
                        out_hbm.at[pl.ds(c * NP + s * ZR, ZR)])

    return scatter


def _upd0_body(aggn_ref, aggf_ref, cn_ref, wnb_ref, wftop_ref, wfb_ref,
               node1_ref, feat1_ref):
    aggn = aggn_ref[0:NP, :] + aggn_ref[NP:2 * NP, :]
    node1_ref[...] = _leaky(cn_ref[...] + _dot(aggn, wnb_ref[...]))
    feat1_ref[...] = _leaky(wftop_ref[...] + _dot(aggf_ref[...], wfb_ref[...]))


def _upd1_body(aggn_ref, aggf_ref, node1_ref, feat1_ref, wnt_ref, wnb_ref,
               wft_ref, wfb_ref, node2_ref, feat2_ref):
    aggn = aggn_ref[0:NP, :] + aggn_ref[NP:2 * NP, :]
    node2_ref[...] = _leaky(_dot(node1_ref[...], wnt_ref[...])
                            + _dot(aggn, wnb_ref[...]))
    feat2_ref[...] = _leaky(_dot(feat1_ref[...], wft_ref[...])
                            + _dot(aggf_ref[...], wfb_ref[...]))


def _upd2_body(aggn_ref, aggf_ref, node2_ref, feat2_ref, wnt_ref, wnb_ref,
               wft_ref, wfb_ref, logits_ref, wk_ref, node3n_ref, bmat_ref,
               probs_ref, kl_ref):
    aggn = aggn_ref[0:NP, :] + aggn_ref[NP:2 * NP, :]
    node3 = _leaky(_dot(node2_ref[...], wnt_ref[...]) + _dot(aggn, wnb_ref[...]))
    nrm = jnp.sqrt(jnp.sum(node3 * node3, axis=1, keepdims=True))
    node3n_ref[...] = node3 / (nrm + 1e-12)
    feat3 = _leaky(_dot(feat2_ref[...], wft_ref[...])
                   + _dot(aggf_ref[...], wfb_ref[...]))
    fnrm = jnp.sqrt(jnp.sum(feat3 * feat3, axis=1, keepdims=True))
    feat3n = feat3 / (fnrm + 1e-12)
    probs = jax.nn.sigmoid(logits_ref[...] / TAU)
    probs_ref[...] = probs
    kl = probs * jnp.log(probs / 0.5 + 1e-12) \
        + (1.0 - probs) * jnp.log((1.0 - probs) / 0.5 + 1e-12)
    kl_ref[...] = jnp.sum(kl, keepdims=True).reshape(1, 1) / (F * F)
    ctx = _dot(probs, feat3n)
    bmat_ref[...] = _dot(ctx, wk_ref[...])


NROW = 2000


def _head_body(node3n_ref, x_ref, xc_ref, mask_ref, wq_ref, bmat_ref, rb_ref,
               nw_ref, nb_ref, dhat_ref, adj_ref, y_ref):
    q = _dot(node3n_ref[...], wq_ref[...])
    scale = 1.0 / (HEADS * jnp.sqrt(jnp.float32(NE // HEADS)))
    d = lax.dot_general(q, bmat_ref[...], (((1,), (1,)), ((), ())),
                        preferred_element_type=_f32) * scale + rb_ref[...]
    dhat_ref[...] = d
    adj_ref[...] = jnp.where(mask_ref[...] == 1, x_ref[...], d)
    y_ref[...] = _dot(xc_ref[...], nw_ref[...]) + nb_ref[...]


def _bf16r(a):
    return a.astype(jnp.bfloat16).astype(_f32)


def kernel(x, x_complete, mask, edge_src, edge_dst, edge_value, params):
    p = params

    pad = EP - E_REAL
    src_pad = jnp.concatenate(
        [edge_src, jnp.full((pad,), N_REAL, jnp.int32)]).astype(jnp.int32)
    dst_pad = jnp.concatenate([edge_dst, jnp.zeros((pad,), jnp.int32)])
    ev_pad = jnp.concatenate([edge_value, jnp.zeros((pad,), _f32)])
    src2d = src_pad.reshape(EP // 128, 128)
    dst_col = dst_pad.reshape(EP, 1)
    ev_col = _bf16r(ev_pad.reshape(EP, 1))
    zeros_np = jnp.zeros((NP, MSG), _f32)

    wm1_0, wm2_0, wn_0, wf_0, we_0 = (p["b0_Wm1"], p["b0_Wm2"], p["b0_Wn"],
                                      p["b0_Wf"], p["b0_We"])
    rows0 = jnp.zeros((8, MSG), _f32)
    rows0 = rows0.at[0].set(_bf16r(wm1_0[F]))
    rows0 = rows0.at[1].set(_bf16r(wm2_0[F]))
    rows0 = rows0.at[2].set(_bf16r(we_0[2 * F]))
    rows0 = rows0.at[3].set(_bf16r(wm2_0[:F]).sum(0))
    rows0 = rows0.at[4].set(_bf16r(we_0[:F]).sum(0))
    cn0 = _bf16r(wn_0[:F]).sum(0).reshape(1, NE)

    wm1_1, wm2_1, wn_1, wf_1, we_1 = (p["b1_Wm1"], p["b1_Wm2"], p["b1_Wn"],
                                      p["b1_Wf"], p["b1_We"])
    wm1_2, wm2_2, wn_2, wf_2 = (p["b2_Wm1"], p["b2_Wm2"], p["b2_Wn"],
                                p["b2_Wf"])

    sc_gather = _make_sc_gather(NE)
    sc_scatter = _make_sc_scatter()

    blk0 = _edge_chunk_call(
        _blk0_body, [1, 1], [(F, MSG), (F, MSG), (8, MSG)],
        [(EP, MSG), (EP, MSG), (F, MSG)], [True, True, False])
    m1_0, e1, aggf0 = blk0(dst_col, ev_col, _bf16r(wm1_0[:F]),
                           _bf16r(we_0[F:2 * F]), rows0)

    aggn0 = sc_scatter(src2d, m1_0, zeros_np)

    upd0 = pl.pallas_call(
        _upd0_body,
        out_shape=[
            jax.ShapeDtypeStruct((NP, NE), _f32),
            jax.ShapeDtypeStruct((F, NE), _f32),
        ],
    )
    node1, feat1 = upd0(aggn0, aggf0, cn0, wn_0[F:], _bf16r(wf_0[:F]),
                        wf_0[F:])

    g1 = sc_gather(node1, src2d)

    blk1 = _edge_chunk_call(
        _blk1_body, [1, NE, NE],
        [(F, NE), (NE, MSG), (NE, MSG), (NE, MSG), (NE, MSG), (NE, NE),
         (NE, NE), (NE, NE)],
        [(EP, MSG), (EP, NE), (F, MSG)], [True, True, False])
    m1_1, e2, aggf1 = blk1(dst_col, e1, g1, feat1, wm1_1[:NE], wm1_1[NE:],
                           wm2_1[:NE], wm2_1[NE:], we_1[:NE],
                           we_1[NE:2 * NE], we_1[2 * NE:])

    aggn1 = sc_scatter(src2d, m1_1, zeros_np)

    upd1 = pl.pallas_call(
        _upd1_body,
        out_shape=[
            jax.ShapeDtypeStruct((NP, NE), _f32),
            jax.ShapeDtypeStruct((F, NE), _f32),
        ],
    )
    node2, feat2 = upd1(aggn1, aggf1, node1, feat1, wn_1[:NE], wn_1[NE:],
                        wf_1[:NE], wf_1[NE:])

    g2 = sc_gather(node2, src2d)

    blk2 = _edge_chunk_call(
        _blk2_body, [1, NE, NE],
        [(F, NE), (NE, MSG), (NE, MSG), (NE, MSG), (NE, MSG)],
        [(EP, MSG), (F, MSG)], [True, False])
    m1_2, aggf2 = blk2(dst_col, e2, g2, feat2, wm1_2[:NE], wm1_2[NE:],
                       wm2_2[:NE], wm2_2[NE:])

    aggn2 = sc_scatter(src2d, m1_2, zeros_np)

    upd2 = pl.pallas_call(
        _upd2_body,
        out_shape=[
            jax.ShapeDtypeStruct((NP, NE), _f32),
            jax.ShapeDtypeStruct((F, NE), _f32),
            jax.ShapeDtypeStruct((F, F), _f32),
            jax.ShapeDtypeStruct((1, 1), _f32),
        ],
    )
    node3n, bmat, probs, kl = upd2(
        aggn2, aggf2, node2, feat2, wn_2[:NE], wn_2[NE:], wf_2[:NE],
        wf_2[NE:], p["gll_logits"], p["reph_Wk"])

    head = pl.pallas_call(
        _head_body,
        grid=(N_REAL // NROW,),
        in_specs=[
            pl.BlockSpec((NROW, NE), lambda i: (i, 0)),
            pl.BlockSpec((NROW, F), lambda i: (i, 0)),
            pl.BlockSpec((NROW, F), lambda i: (i, 0)),
            pl.BlockSpec((NROW, F), lambda i: (i, 0)),
            pl.BlockSpec((NE, NE), lambda i: (0, 0)),
            pl.BlockSpec((F, NE), lambda i: (0, 0)),
            pl.BlockSpec((1, F), lambda i: (0, 0)),
            pl.BlockSpec((F, 10), lambda i: (0, 0)),
            pl.BlockSpec((1, 10), lambda i: (0, 0)),
        ],
        out_specs=[
            pl.BlockSpec((NROW, F), lambda i: (i, 0)),
            pl.BlockSpec((NROW, F), lambda i: (i, 0)),
            pl.BlockSpec((NROW, 10), lambda i: (i, 0)),
        ],
        out_shape=[
            jax.ShapeDtypeStruct((N_REAL, F), _f32),
            jax.ShapeDtypeStruct((N_REAL, F), _f32),
            jax.ShapeDtypeStruct((N_REAL, 10), _f32),
        ],
    )
    d_hat, d_hat_adj, y_hat = head(
        node3n[:N_REAL], x, x_complete, mask, p["reph_Wq"], bmat,
        p["reph_b"].reshape(1, F), p["nph_W"], p["nph_b"].reshape(1, 10))

    return d_hat, d_hat_adj, y_hat, kl.reshape(()), probs


# speedup vs baseline: 1.2396x; 1.2396x over previous
import functools

import jax
import jax.numpy as jnp
from jax import lax
from jax.experimental import pallas as pl
from jax.experimental.pallas import tpu as pltpu
from jax.experimental.pallas import tpu_sc as plsc

F = 128
NE = 64
MSG = 64
HEADS = 2
TAU = 0.1
N_REAL = 10000
E_REAL = 320000

NP = 10240
CE = 4096
NCHUNK = 80
EP = NCHUNK * CE
NT = 32
ET = EP // NT
KT = ET // 128
ZR = NP // 16

_f32 = jnp.float32
_HI = lax.Precision.HIGHEST


def _leaky(v):
    return jnp.where(v >= 0, v, 0.01 * v)


def _onehot(dst, n_rows):
    return (dst == lax.broadcasted_iota(jnp.int32, (n_rows, F), 1)).astype(_f32)


def _dot(a, b):
    return jnp.dot(a, b, preferred_element_type=_f32)


def _doth(a, b):
    return jnp.dot(a, b, preferred_element_type=_f32, precision=_HI)


def _blk0_body(dst_ref, ev_ref, wm1f_ref, wef_ref, rows_ref, m1_ref, e1_ref,
               aggf_ref):
    i = pl.program_id(0)
    dst = dst_ref[...]
    ev = ev_ref[...]
    oh = _onehot(dst, CE)
    rows = rows_ref[...]
    w1, w2, we, c_m2, c_e = (rows[0:1], rows[1:2], rows[2:3], rows[3:4],
                             rows[4:5])
    m1 = jnp.maximum(_dot(oh, wm1f_ref[...]) + ev * w1, 0.0)
    m1_ref[...] = m1
    e1 = c_e + _dot(oh, wef_ref[...]) + ev * we
    e1_ref[...] = _leaky(e1)
    m2 = jnp.maximum(c_m2 + ev * w2, 0.0)
    gidx = i * CE + lax.broadcasted_iota(jnp.int32, (CE, 1), 0)
    m2 = jnp.where(gidx < E_REAL, m2, 0.0)
    part = _split_segsum(oh, m2)

    @pl.when(i == 0)
    def _():
        aggf_ref[...] = jnp.zeros_like(aggf_ref)

    aggf_ref[...] += part


def _split_gather(oh, tbl):
    hi = tbl.astype(jnp.bfloat16).astype(_f32)
    lo = tbl - hi
    return _dot(oh, hi) + _dot(oh, lo)


def _split_segsum(oh, vals):
    hi = vals.astype(jnp.bfloat16).astype(_f32)
    lo = vals - hi
    dn = (((0,), (0,)), ((), ()))
    return (lax.dot_general(oh, hi, dn, preferred_element_type=_f32)
            + lax.dot_general(oh, lo, dn, preferred_element_type=_f32))


def _blk1_body(dst_ref, e1_ref, g_ref, feat_ref, wm1t_ref, wm1b_ref, wm2t_ref,
               wm2b_ref, wet_ref, wef_ref, web_ref, m1_ref, e2_ref, aggf_ref):
    i = pl.program_id(0)
    dst = dst_ref[...]
    e1 = e1_ref[...]
    g = g_ref[...]
    oh = _onehot(dst, CE)
    featd = _split_gather(oh, feat_ref[...])
    m1 = jnp.maximum(_dot(featd, wm1t_ref[...]) + _dot(e1, wm1b_ref[...]), 0.0)
    m1_ref[...] = m1
    m2 = jnp.maximum(_dot(g, wm2t_ref[...]) + _dot(e1, wm2b_ref[...]), 0.0)
    e2 = _dot(g, wet_ref[...]) + _dot(featd, wef_ref[...]) \
        + _dot(e1, web_ref[...])
    e2_ref[...] = _leaky(e2)
    gidx = i * CE + lax.broadcasted_iota(jnp.int32, (CE, 1), 0)
    m2 = jnp.where(gidx < E_REAL, m2, 0.0)
    part = _split_segsum(oh, m2)

    @pl.when(i == 0)
    def _():
        aggf_ref[...] = jnp.zeros_like(aggf_ref)

    aggf_ref[...] += part


def _blk2_body(dst_ref, e2_ref, g2_ref, feat_ref, wm1t_ref, wm1b_ref,
               wm2t_ref, wm2b_ref, m1_ref, aggf_ref):
    i = pl.program_id(0)
    dst = dst_ref[...]
    e2 = e2_ref[...]
    oh = _onehot(dst, CE)
    featd = _split_gather(oh, feat_ref[...])
    m1 = jnp.maximum(_dot(featd, wm1t_ref[...]) + _dot(e2, wm1b_ref[...]), 0.0)
    m1_ref[...] = m1
    m2 = jnp.maximum(_dot(g2_ref[...], wm2t_ref[...])
                     + _dot(e2, wm2b_ref[...]), 0.0)
    gidx = i * CE + lax.broadcasted_iota(jnp.int32, (CE, 1), 0)
    m2 = jnp.where(gidx < E_REAL, m2, 0.0)
    part = _split_segsum(oh, m2)

    @pl.when(i == 0)
    def _():
        aggf_ref[...] = jnp.zeros_like(aggf_ref)

    aggf_ref[...] += part


def _edge_chunk_call(body, chunk_shapes, const_shapes, out_shapes,
                     out_chunked):
    in_specs = []
    for shp in chunk_shapes:
        in_specs.append(pl.BlockSpec((CE, shp), lambda i: (i, 0)))
    for shp in const_shapes:
        in_specs.append(
            pl.BlockSpec(shp, lambda i, _z=tuple(0 for _ in shp): _z))
    out_specs = []
    for shp, chunked in zip(out_shapes, out_chunked):
        if chunked:
            out_specs.append(pl.BlockSpec((CE, shp[1]), lambda i: (i, 0)))
        else:
            out_specs.append(pl.BlockSpec(shp, lambda i: (0, 0)))
    out_shape = [
        jax.ShapeDtypeStruct((EP, shp[1]) if chunked else shp, _f32)
        for shp, chunked in zip(out_shapes, out_chunked)
    ]
    return pl.pallas_call(
        body,
        grid=(NCHUNK,),
        in_specs=in_specs,
        out_specs=out_specs,
        out_shape=out_shape,
    )


def _sc_mesh():
    return plsc.VectorSubcoreMesh(core_axis_name="c", subcore_axis_name="s")


def _make_sc_gather(width):
    @functools.partial(
        pl.kernel,
        out_type=jax.ShapeDtypeStruct((EP, width), _f32),
        mesh=_sc_mesh(),
        compiler_params=pltpu.CompilerParams(use_tc_tiling_on_sc=False),
        scratch_types=[
            pltpu.VMEM((KT, 128), jnp.int32),
            pltpu.VMEM((128, width), _f32),
            pltpu.VMEM((128, width), _f32),
            pltpu.SemaphoreType.DMA,
            pltpu.SemaphoreType.DMA,
        ],
    )
    def gather(table_hbm, idx_hbm, out_hbm, idx_v, b0, b1, s0, s1):
        c = lax.axis_index("c")
        s = lax.axis_index("s")
        t = c * 16 + s
        base = t * KT
        pltpu.sync_copy(idx_hbm.at[pl.ds(base, KT)], idx_v)
        pltpu.async_copy(table_hbm.at[idx_v.at[0]], b0, s0)

        def body(h, carry):
            j0 = 2 * h
            d1 = pltpu.async_copy(table_hbm.at[idx_v.at[j0 + 1]], b1, s1)
            pltpu.make_async_copy(table_hbm.at[idx_v.at[j0]], b0, s0).wait()
            pltpu.sync_copy(b0, out_hbm.at[pl.ds((base + j0) * 128, 128)])

            @pl.when(h + 1 < KT // 2)
            def _():
                pltpu.async_copy(table_hbm.at[idx_v.at[j0 + 2]], b0, s0)

            d1.wait()
            pltpu.sync_copy(b1, out_hbm.at[pl.ds((base + j0 + 1) * 128, 128)])
            return carry

        lax.fori_loop(0, KT // 2, body, 0)

    return gather


def _make_sc_scatter():
    @functools.partial(
        pl.kernel,
        out_type=jax.ShapeDtypeStruct((2 * NP, MSG), _f32),
        mesh=_sc_mesh(),
        compiler_params=pltpu.CompilerParams(use_tc_tiling_on_sc=False),
        scratch_types=[
            pltpu.VMEM((KT, 128), jnp.int32),
            pltpu.VMEM((128, MSG), _f32),
            pltpu.VMEM((128, MSG), _f32),
            pltpu.VMEM_SHARED((NP, MSG), _f32),
            pltpu.SemaphoreType.DMA,
            pltpu.SemaphoreType.DMA,
        ],
    )
    def scatter(idx_hbm, vals_hbm, zeros_hbm, out_hbm, idx_v, v0, v1, acc,
                s0, s1):
        c = lax.axis_index("c")
        s = lax.axis_index("s")
        t = c * 16 + s
        pltpu.sync_copy(zeros_hbm.at[pl.ds(s * ZR, ZR)], acc.at[pl.ds(s * ZR, ZR)])
        plsc.subcore_barrier()
        base = t * KT
        pltpu.sync_copy(idx_hbm.at[pl.ds(base, KT)], idx_v)
        pltpu.async_copy(vals_hbm.at[pl.ds(base * 128, 128)], v0, s0)

        def body(h, carry):
            j0 = 2 * h
            d1 = pltpu.async_copy(vals_hbm.at[pl.ds((base + j0 + 1) * 128, 128)],
                                  v1, s1)
            pltpu.make_async_copy(vals_hbm.at[pl.ds(base * 128, 128)], v0,
                                  s0).wait()
            pltpu.sync_copy(v0, acc.at[idx_v.at[j0]], add=True)

            @pl.when(h + 1 < KT // 2)
            def _():
                pltpu.async_copy(vals_hbm.at[pl.ds((base + j0 + 2) * 128, 128)],
                                 v0, s0)

            d1.wait()
            pltpu.sync_copy(v1, acc.at[idx_v.at[j0 + 1]], add=True)
            return carry

        lax.fori_loop(0, KT // 2, body, 0)
        plsc.subcore_barrier()
        pltpu.sync_copy(acc.at[pl.ds(s * ZR, ZR)],
                        out_hbm.at[pl.ds(c * NP + s * ZR, ZR)])

    return scatter


def _upd0_body(aggn_ref, aggf_ref, cn_ref, wnb_ref, wftop_ref, wfb_ref,
               node1_ref, feat1_ref):
    aggn = aggn_ref[0:NP, :] + aggn_ref[NP:2 * NP, :]
    node1_ref[...] = _leaky(cn_ref[...] + _dot(aggn, wnb_ref[...]))
    feat1_ref[...] = _leaky(wftop_ref[...] + _dot(aggf_ref[...], wfb_ref[...]))


def _upd1_body(aggn_ref, aggf_ref, node1_ref, feat1_ref, wnt_ref, wnb_ref,
               wft_ref, wfb_ref, node2_ref, feat2_ref):
    aggn = aggn_ref[0:NP, :] + aggn_ref[NP:2 * NP, :]
    node2_ref[...] = _leaky(_dot(node1_ref[...], wnt_ref[...])
                            + _dot(aggn, wnb_ref[...]))
    feat2_ref[...] = _leaky(_dot(feat1_ref[...], wft_ref[...])
                            + _dot(aggf_ref[...], wfb_ref[...]))


def _upd2_body(aggn_ref, aggf_ref, node2_ref, feat2_ref, wnt_ref, wnb_ref,
               wft_ref, wfb_ref, logits_ref, wk_ref, node3n_ref, bmat_ref,
               probs_ref, kl_ref):
    aggn = aggn_ref[0:NP, :] + aggn_ref[NP:2 * NP, :]
    node3 = _leaky(_dot(node2_ref[...], wnt_ref[...]) + _dot(aggn, wnb_ref[...]))
    nrm = jnp.sqrt(jnp.sum(node3 * node3, axis=1, keepdims=True))
    node3n_ref[...] = node3 / (nrm + 1e-12)
    feat3 = _leaky(_dot(feat2_ref[...], wft_ref[...])
                   + _dot(aggf_ref[...], wfb_ref[...]))
    fnrm = jnp.sqrt(jnp.sum(feat3 * feat3, axis=1, keepdims=True))
    feat3n = feat3 / (fnrm + 1e-12)
    probs = jax.nn.sigmoid(logits_ref[...] / TAU)
    probs_ref[...] = probs
    kl = probs * jnp.log(probs / 0.5 + 1e-12) \
        + (1.0 - probs) * jnp.log((1.0 - probs) / 0.5 + 1e-12)
    kl_ref[...] = jnp.sum(kl, keepdims=True).reshape(1, 1) / (F * F)
    ctx = _dot(probs, feat3n)
    bmat_ref[...] = _dot(ctx, wk_ref[...])


NROW = 2000


def _head_body(node3n_ref, x_ref, xc_ref, mask_ref, wq_ref, bmat_ref, rb_ref,
               nw_ref, nb_ref, dhat_ref, adj_ref, y_ref):
    q = _dot(node3n_ref[...], wq_ref[...])
    scale = 1.0 / (HEADS * jnp.sqrt(jnp.float32(NE // HEADS)))
    d = lax.dot_general(q, bmat_ref[...], (((1,), (1,)), ((), ())),
                        preferred_element_type=_f32) * scale + rb_ref[...]
    dhat_ref[...] = d
    adj_ref[...] = jnp.where(mask_ref[...] == 1, x_ref[...], d)
    y_ref[...] = _dot(xc_ref[...], nw_ref[...]) + nb_ref[...]


def _bf16r(a):
    return a.astype(jnp.bfloat16).astype(_f32)


def kernel(x, x_complete, mask, edge_src, edge_dst, edge_value, params):
    p = params

    pad = EP - E_REAL
    src_pad = jnp.concatenate(
        [edge_src, jnp.full((pad,), N_REAL, jnp.int32)]).astype(jnp.int32)
    dst_pad = jnp.concatenate([edge_dst, jnp.zeros((pad,), jnp.int32)])
    ev_pad = jnp.concatenate([edge_value, jnp.zeros((pad,), _f32)])
    src2d = src_pad.reshape(EP // 128, 128)
    dst_col = dst_pad.reshape(EP, 1)
    ev_col = _bf16r(ev_pad.reshape(EP, 1))
    zeros_np = jnp.zeros((NP, MSG), _f32)

    wm1_0, wm2_0, wn_0, wf_0, we_0 = (p["b0_Wm1"], p["b0_Wm2"], p["b0_Wn"],
                                      p["b0_Wf"], p["b0_We"])
    rows0 = jnp.zeros((8, MSG), _f32)
    rows0 = rows0.at[0].set(_bf16r(wm1_0[F]))
    rows0 = rows0.at[1].set(_bf16r(wm2_0[F]))
    rows0 = rows0.at[2].set(_bf16r(we_0[2 * F]))
    rows0 = rows0.at[3].set(_bf16r(wm2_0[:F]).sum(0))
    rows0 = rows0.at[4].set(_bf16r(we_0[:F]).sum(0))
    cn0 = _bf16r(wn_0[:F]).sum(0).reshape(1, NE)

    wm1_1, wm2_1, wn_1, wf_1, we_1 = (p["b1_Wm1"], p["b1_Wm2"], p["b1_Wn"],
                                      p["b1_Wf"], p["b1_We"])
    wm1_2, wm2_2, wn_2, wf_2 = (p["b2_Wm1"], p["b2_Wm2"], p["b2_Wn"],
                                p["b2_Wf"])

    sc_gather = _make_sc_gather(NE)
    sc_scatter = _make_sc_scatter()

    blk0 = _edge_chunk_call(
        _blk0_body, [1, 1], [(F, MSG), (F, MSG), (8, MSG)],
        [(EP, MSG), (EP, MSG), (F, MSG)], [True, True, False])
    m1_0, e1, aggf0 = blk0(dst_col, ev_col, _bf16r(wm1_0[:F]),
                           _bf16r(we_0[F:2 * F]), rows0)

    aggn0 = sc_scatter(src2d, m1_0, zeros_np)

    upd0 = pl.pallas_call(
        _upd0_body,
        out_shape=[
            jax.ShapeDtypeStruct((NP, NE), _f32),
            jax.ShapeDtypeStruct((F, NE), _f32),
        ],
    )
    node1, feat1 = upd0(aggn0, aggf0, cn0, wn_0[F:], _bf16r(wf_0[:F]),
                        wf_0[F:])

    g1 = sc_gather(node1, src2d)

    blk1 = _edge_chunk_call(
        _blk1_body, [1, NE, NE],
        [(F, NE), (NE, MSG), (NE, MSG), (NE, MSG), (NE, MSG), (NE, NE),
         (NE, NE), (NE, NE)],
        [(EP, MSG), (EP, NE), (F, MSG)], [True, True, False])
    m1_1, e2, aggf1 = blk1(dst_col, e1, g1, feat1, wm1_1[:NE], wm1_1[NE:],
                           wm2_1[:NE], wm2_1[NE:], we_1[:NE],
                           we_1[NE:2 * NE], we_1[2 * NE:])

    aggn1 = sc_scatter(src2d, m1_1, zeros_np)

    upd1 = pl.pallas_call(
        _upd1_body,
        out_shape=[
            jax.ShapeDtypeStruct((NP, NE), _f32),
            jax.ShapeDtypeStruct((F, NE), _f32),
        ],
    )
    node2, feat2 = upd1(aggn1, aggf1, node1, feat1, wn_1[:NE], wn_1[NE:],
                        wf_1[:NE], wf_1[NE:])

    g2 = sc_gather(node2, src2d)

    blk2 = _edge_chunk_call(
        _blk2_body, [1, NE, NE],
        [(F, NE), (NE, MSG), (NE, MSG), (NE, MSG), (NE, MSG)],
        [(EP, MSG), (F, MSG)], [True, False])
    m1_2, aggf2 = blk2(dst_col, e2, g2, feat2, wm1_2[:NE], wm1_2[NE:],
                       wm2_2[:NE], wm2_2[NE:])

    aggn2 = sc_scatter(src2d, m1_2, zeros_np)

    upd2 = pl.pallas_call(
        _upd2_body,
        out_shape=[
            jax.ShapeDtypeStruct((NP, NE), _f32),
            jax.ShapeDtypeStruct((F, NE), _f32),
            jax.ShapeDtypeStruct((F, F), _f32),
            jax.ShapeDtypeStruct((1, 1), _f32),
        ],
    )
    node3n, bmat, probs, kl = upd2(
        aggn2, aggf2, node2, feat2, wn_2[:NE], wn_2[NE:], wf_2[:NE],
        wf_2[NE:], p["gll_logits"], p["reph_Wk"])

    head = pl.pallas_call(
        _head_body,
        grid=(N_REAL // NROW,),
        in_specs=[
            pl.BlockSpec((NROW, NE), lambda i: (i, 0)),
            pl.BlockSpec((NROW, F), lambda i: (i, 0)),
            pl.BlockSpec((NROW, F), lambda i: (i, 0)),
            pl.BlockSpec((NROW, F), lambda i: (i, 0)),
            pl.BlockSpec((NE, NE), lambda i: (0, 0)),
            pl.BlockSpec((F, NE), lambda i: (0, 0)),
            pl.BlockSpec((1, F), lambda i: (0, 0)),
            pl.BlockSpec((F, 10), lambda i: (0, 0)),
            pl.BlockSpec((1, 10), lambda i: (0, 0)),
        ],
        out_specs=[
            pl.BlockSpec((NROW, F), lambda i: (i, 0)),
            pl.BlockSpec((NROW, F), lambda i: (i, 0)),
            pl.BlockSpec((NROW, 10), lambda i: (i, 0)),
        ],
        out_shape=[
            jax.ShapeDtypeStruct((N_REAL, F), _f32),
            jax.ShapeDtypeStruct((N_REAL, F), _f32),
            jax.ShapeDtypeStruct((N_REAL, 10), _f32),
        ],
    )
    d_hat, d_hat_adj, y_hat = head(
        node3n[:N_REAL], x, x_complete, mask, p["reph_Wq"], bmat,
        p["reph_b"].reshape(1, F), p["nph_W"], p["nph_b"].reshape(1, 10))

    return d_hat, d_hat_adj, y_hat, kl.reshape(()), probs
